# baseline (speedup 1.0000x reference)
import jax
import jax.numpy as jnp
import numpy as np
from jax import lax
from jax.experimental import pallas as pl
from jax.experimental.pallas import tpu as pltpu
from jax.experimental.pallas import tpu_sc as plsc

N = 10000
E = 320000
H = 128

NC = 2
NS = 16
NW = NC * NS
EW = E // NW
EWP = 10240
EP = NW * EWP
C = 128
NCH = EWP // C
GRP = 2
NGRP = NCH // GRP
GE = GRP * C
NP = 10112

_F32 = jnp.float32
_I32 = jnp.int32


def _mesh():
    return plsc.VectorSubcoreMesh(
        core_axis_name="c", subcore_axis_name="s", num_cores=NC, num_subcores=NS
    )


def _wid():
    return lax.axis_index("s") * NC + lax.axis_index("c")


_CP = pltpu.CompilerParams(needs_layout_passes=False)


def _sc_gather_one(table, idx3d):
    def body(tab, idx, out, idxv, buf_a, buf_b, gsem_a, gsem_b, osem_a, osem_b):
        wid = _wid()
        pltpu.sync_copy(idx.at[wid], idxv)
        base = wid * EWP

        def fire(gi, buf, sem):
            for j in range(GRP):
                pltpu.async_copy(
                    tab.at[idxv.at[gi * GRP + j]], buf.at[pl.ds(j * C, C)], sem
                )

        def drain(buf, sem):
            pltpu.make_async_copy(tab.at[pl.ds(0, GE)], buf, sem).wait()

        fire(0, buf_a, gsem_a)

        def outer(go, carry):
            gi0 = 2 * go
            gi1 = 2 * go + 1

            @pl.when(go > 0)
            def _():
                drain(buf_b, osem_b)

            @pl.when(gi1 < NGRP)
            def _():
                fire(gi1, buf_b, gsem_b)

            drain(buf_a, gsem_a)
            pltpu.async_copy(buf_a, out.at[pl.ds(base + gi0 * GE, GE)], osem_a)

            @pl.when(gi0 + 2 < NGRP)
            def _():
                drain(buf_a, osem_a)
                fire(gi0 + 2, buf_a, gsem_a)

            @pl.when(gi1 < NGRP)
            def _():
                drain(buf_b, gsem_b)
                pltpu.async_copy(buf_b, out.at[pl.ds(base + gi1 * GE, GE)], osem_b)

            return carry

        lax.fori_loop(0, (NGRP + 1) // 2, outer, 0)
        drain(buf_a, osem_a)

    fn = pl.kernel(
        body,
        out_type=jax.ShapeDtypeStruct((EP, H), _F32),
        mesh=_mesh(),
        compiler_params=_CP,
        scratch_types=[
            pltpu.VMEM((NCH, C), _I32),
            pltpu.VMEM((GE, H), _F32),
            pltpu.VMEM((GE, H), _F32),
            pltpu.SemaphoreType.DMA,
            pltpu.SemaphoreType.DMA,
            pltpu.SemaphoreType.DMA,
            pltpu.SemaphoreType.DMA,
        ],
    )
    return fn(table, idx3d)


_RQ = 2560
_RQCH = _RQ // C


def _sc_rel(pos_t, row3d, col3d):
    def body(pt, ridx, cidx, out, tab, idxr, idxc, ob):
        wid = _wid()
        pltpu.sync_copy(pt, tab)
        pltpu.sync_copy(ridx.at[wid], idxr)
        pltpu.sync_copy(cidx.at[wid], idxc)

        zeros16 = jnp.zeros((16,), _F32)
        k3 = jnp.full((16,), 3, _I32)

        def quarter(q, carry):
            for l in range(_RQCH):
                i = q * _RQCH + l
                ii = jnp.full((16,), i, _I32)
                for j in range(C // 16):
                    r = lax.iota(_I32, 16) + j * 16
                    ir = plsc.load_gather(idxr, [ii, r])
                    ic = plsc.load_gather(idxc, [ii, r])
                    rl = r + l * C
                    for k in range(3):
                        kk = jnp.full((16,), k, _I32)
                        pk = plsc.load_gather(tab, [kk, ir])
                        qk = plsc.load_gather(tab, [kk, ic])
                        plsc.store_scatter(ob, [kk, rl], qk - pk)
                    plsc.store_scatter(ob, [k3, rl], zeros16)
            pltpu.sync_copy(ob, out.at[wid, :, pl.ds(q * _RQ, _RQ)])
            return carry

        lax.fori_loop(0, EWP // _RQ, quarter, 0)

    fn = pl.kernel(
        body,
        out_type=jax.ShapeDtypeStruct((NW, 4, EWP), _F32),
        mesh=_mesh(),
        compiler_params=_CP,
        scratch_types=[
            pltpu.VMEM((4, N), _F32),
            pltpu.VMEM((NCH, C), _I32),
            pltpu.VMEM((NCH, C), _I32),
            pltpu.VMEM((4, _RQ), _F32),
        ],
    )
    return fn(pos_t, row3d, col3d)


def _sc_scatter(coef_t, frame_t, col3d, zeros4n):
    def body(coef_h, frame_h, cidx, z4, out, idxb, cb, fb, acc):
        wid = _wid()
        pltpu.sync_copy(z4, acc)
        pltpu.sync_copy(cidx.at[wid], idxb)

        k0 = jnp.full((16,), 0, _I32)
        k1 = jnp.full((16,), 1, _I32)
        k2 = jnp.full((16,), 2, _I32)

        def chunk(i, carry):
            pltpu.sync_copy(coef_h.at[wid, :, pl.ds(i * C, C)], cb)
            pltpu.sync_copy(frame_h.at[wid, :, pl.ds(i * C, C)], fb)
            ii = jnp.full((16,), i, _I32)
            for j in range(C // 16):
                r = lax.iota(_I32, 16) + j * 16
                ids = plsc.load_gather(idxb, [ii, r])

                def ldf(k):
                    return plsc.load_gather(fb, [jnp.full((16,), k, _I32), r])

                fs = plsc.load_gather(cb, [k0, r])
                fv0 = plsc.load_gather(cb, [k1, r])
                fv1 = plsc.load_gather(cb, [k2, r])
                e1x, e1y, e1z = ldf(4), ldf(5), ldf(6)
                e2x, e2y = ldf(8), ldf(9)
                e3x, e3y, e3z = ldf(12), ldf(13), ldf(14)
                fx = fs * e1x + fv0 * e2x + fv1 * e3x
                fy = fs * e1y + fv0 * e2y + fv1 * e3y
                fz = fs * e1z + fv1 * e3z
                plsc.addupdate_scatter(acc, [k0, ids], fx)
                plsc.addupdate_scatter(acc, [k1, ids], fy)
                plsc.addupdate_scatter(acc, [k2, ids], fz)
            return carry

        lax.fori_loop(0, NCH, chunk, 0)
        pltpu.sync_copy(acc, out.at[wid])

    fn = pl.kernel(
        body,
        out_type=jax.ShapeDtypeStruct((NW, 4, NP), _F32),
        mesh=_mesh(),
        compiler_params=_CP,
        scratch_types=[
            pltpu.VMEM((NCH, C), _I32),
            pltpu.VMEM((4, C), _F32),
            pltpu.VMEM((16, C), _F32),
            pltpu.VMEM((4, NP), _F32),
        ],
    )
    return fn(coef_t, frame_t, col3d, zeros4n)


def _mlp2_body(x_ref, w1_ref, b1_ref, w2_ref, b2_ref, o_ref):
    t = jnp.dot(x_ref[...], w1_ref[...], preferred_element_type=_F32) + b1_ref[...]
    t = jnp.maximum(t, 0.0)
    o_ref[...] = jnp.dot(t, w2_ref[...], preferred_element_type=_F32) + b2_ref[...]


def _tc_mlp2(x, w1, b1, w2, b2, bn):
    n, fi = x.shape
    fh = w1.shape[1]
    fo = w2.shape[1]
    return pl.pallas_call(
        _mlp2_body,
        grid=(n // bn,),
        in_specs=[
            pl.BlockSpec((bn, fi), lambda i: (i, 0)),
            pl.BlockSpec((fi, fh), lambda i: (0, 0)),
            pl.BlockSpec((1, fh), lambda i: (0, 0)),
            pl.BlockSpec((fh, fo), lambda i: (0, 0)),
            pl.BlockSpec((1, fo), lambda i: (0, 0)),
        ],
        out_specs=pl.BlockSpec((bn, fo), lambda i: (i, 0)),
        out_shape=jax.ShapeDtypeStruct((n, fo), _F32),
    )(x, w1, b1.reshape(1, -1), w2, b2.reshape(1, -1))


_BC = 2560
_NB = EWP // _BC


def _frame_body(rel_ref, ones_ref, s2_ref, pt_ref, r1t_ref, r2t_ref, o_ref):
    f32 = _F32
    rel = rel_ref[0]
    ones44 = ones_ref[...]
    s2 = s2_ref[...]
    ptj = pt_ref[...]
    r1t = r1t_ref[...]
    r2t = r2t_ref[...]

    d2 = jnp.dot(ones44, rel * rel, preferred_element_type=f32)
    e1 = rel / (jnp.sqrt(d2) + 1e-6)
    n2 = jnp.dot(s2, e1 * e1, preferred_element_type=f32)
    e2 = jnp.dot(ptj, e1, preferred_element_type=f32) / (jnp.sqrt(n2) + 1e-6)
    e3 = jnp.dot(r1t, e1, preferred_element_type=f32) * jnp.dot(
        r2t, e2, preferred_element_type=f32
    ) - jnp.dot(r2t, e1, preferred_element_type=f32) * jnp.dot(
        r1t, e2, preferred_element_type=f32
    )
    o_ref[0] = jnp.concatenate([rel, e1, e2, e3], axis=0)


def _tc_frame(rel_t):
    ones44 = jnp.ones((4, 4), _F32)
    s2 = np.zeros((4, 4), np.float32)
    s2[:, 0] = 1.0
    s2[:, 1] = 1.0
    perm = np.zeros((4, 4), np.float32)
    perm[1, 0] = 1.0
    perm[0, 1] = -1.0
    r1 = np.zeros((4, 4), np.float32)
    r1[1, 0] = 1.0
    r1[2, 1] = 1.0
    r1[0, 2] = 1.0
    r2 = np.zeros((4, 4), np.float32)
    r2[2, 0] = 1.0
    r2[0, 1] = 1.0
    r2[1, 2] = 1.0
    full = lambda: pl.BlockSpec((4, 4), lambda i: (0, 0))
    return pl.pallas_call(
        _frame_body,
        grid=(NW * _NB,),
        in_specs=[
            pl.BlockSpec((1, 4, _BC), lambda i: (i // _NB, 0, i % _NB)),
            full(), full(), full(), full(), full(),
        ],
        out_specs=pl.BlockSpec((1, 16, _BC), lambda i: (i // _NB, 0, i % _NB)),
        out_shape=jax.ShapeDtypeStruct((NW, 16, EWP), _F32),
    )(
        rel_t, ones44, jnp.asarray(s2), jnp.asarray(perm.T),
        jnp.asarray(r1.T), jnp.asarray(r2.T),
    )


def _edge_body(
    gr_ref, gc_ref, fr_ref, wa_ref, wb_ref, wc_ref, b1_ref,
    w2s_ref, b2s_ref, w2v_ref, b2v_ref, w3a_ref, w3b_ref, b3_ref, o_ref
):
    rel_t = fr_ref[0, 0:4, :]
    t = (
        jnp.dot(gr_ref[...], wa_ref[...], preferred_element_type=_F32)
        + jnp.dot(gc_ref[...], wb_ref[...], preferred_element_type=_F32)
        + lax.dot_general(
            rel_t, wc_ref[...], (((0,), (0,)), ((), ())),
            preferred_element_type=_F32,
        )
        + b1_ref[...]
    )
    t = jnp.maximum(t, 0.0)
    s = jnp.maximum(
        jnp.dot(t[:, :H], w2s_ref[...], preferred_element_type=_F32) + b2s_ref[...], 0.0
    )
    v = jnp.maximum(
        jnp.dot(t[:, H:], w2v_ref[...], preferred_element_type=_F32) + b2v_ref[...], 0.0
    )
    ct = lax.dot_general(
        w3a_ref[...], s, (((0,), (1,)), ((), ())), preferred_element_type=_F32
    ) + lax.dot_general(
        w3b_ref[...], v, (((0,), (1,)), ((), ())), preferred_element_type=_F32
    )
    b3v = b3_ref[...]
    o_ref[0] = ct + b3v[:, 0:1]


def _tc_edge(gr, gc, frame_t, wa, wb, wc4, b1, w2s, b2s, w2v, b2v, w3a, w3b, b3c):
    full = lambda shape: pl.BlockSpec(shape, lambda i: (0,) * len(shape))
    return pl.pallas_call(
        _edge_body,
        grid=(NW * _NB,),
        in_specs=[
            pl.BlockSpec((_BC, H), lambda i: (i, 0)),
            pl.BlockSpec((_BC, H), lambda i: (i, 0)),
            pl.BlockSpec((1, 16, _BC), lambda i: (i // _NB, 0, i % _NB)),
            full((H, 2 * H)),
            full((H, 2 * H)),
            full((4, 2 * H)),
            full((1, 2 * H)),
            full((H, H)),
            full((1, H)),
            full((H, H)),
            full((1, H)),
            full((H, 4)),
            full((H, 4)),
            full((4, 128)),
        ],
        out_specs=pl.BlockSpec((1, 4, _BC), lambda i: (i // _NB, 0, i % _NB)),
        out_shape=jax.ShapeDtypeStruct((NW, 4, EWP), _F32),
    )(
        gr, gc, frame_t, wa, wb, wc4, b1.reshape(1, -1),
        w2s, b2s.reshape(1, -1), w2v, b2v.reshape(1, -1),
        w3a, w3b, b3c,
    )


def _reduce_body(ag_ref, i4_ref, o_ref):
    a = jnp.sum(ag_ref[...], axis=0)
    o_ref[...] = lax.dot_general(
        a, i4_ref[...], (((0,), (0,)), ((), ())), preferred_element_type=_F32
    )


def _tc_reduce_t(aggr):
    return pl.pallas_call(
        _reduce_body,
        grid=(1,),
        in_specs=[
            pl.BlockSpec((NW, 4, NP), lambda i: (0, 0, 0)),
            pl.BlockSpec((4, 4), lambda i: (0, 0)),
        ],
        out_specs=pl.BlockSpec((NP, 4), lambda i: (0, 0)),
        out_shape=jax.ShapeDtypeStruct((NP, 4), _F32),
    )(aggr, jnp.eye(4, dtype=_F32))


def _update_body(h_ref, ag_ref, wh_ref, wa_ref, b1_ref, w2_ref, b2_ref, o_ref):
    t = (
        jnp.dot(h_ref[...], wh_ref[...], preferred_element_type=_F32)
        + jnp.dot(ag_ref[...], wa_ref[...], preferred_element_type=_F32)
        + b1_ref[...]
    )
    t = jnp.maximum(t, 0.0)
    o_ref[...] = h_ref[...] + jnp.dot(t, w2_ref[...], preferred_element_type=_F32) + b2_ref[...]


def _tc_update(h, aggrt, wh, wa4, b1, w2, b2, bn):
    full = lambda shape: pl.BlockSpec(shape, lambda i: (0,) * len(shape))
    return pl.pallas_call(
        _update_body,
        grid=(N // bn,),
        in_specs=[
            pl.BlockSpec((bn, H), lambda i: (i, 0)),
            pl.BlockSpec((bn, 4), lambda i: (i, 0)),
            full((H, H)),
            full((4, H)),
            full((1, H)),
            full((H, H)),
            full((1, H)),
        ],
        out_specs=pl.BlockSpec((bn, H), lambda i: (i, 0)),
        out_shape=jax.ShapeDtypeStruct((N, H), _F32),
    )(h, aggrt, wh, wa4, b1.reshape(1, -1), w2, b2.reshape(1, -1))


_BN = 2000


def kernel(x, edge_index, pos, params):
    bn = _BN

    x8 = jnp.pad(x, ((0, 0), (0, 2)))
    pos_t = jnp.pad(pos.T, ((0, 1), (0, 0)))
    rowp = jnp.pad(edge_index[0].reshape(NW, EW), ((0, 0), (0, EWP - EW)))
    colw = edge_index[1].reshape(NW, EW)
    row3d = rowp.reshape(NW, NCH, C)
    col3d0 = jnp.pad(colw, ((0, 0), (0, EWP - EW))).reshape(NW, NCH, C)
    col3dN = jnp.pad(
        colw, ((0, 0), (0, EWP - EW)), constant_values=N
    ).reshape(NW, NCH, C)
    zeros4n = jnp.zeros((4, NP), _F32)

    (w_e1, b_e1), (w_e2, b_e2) = params["enc"]
    w_e1p = jnp.pad(w_e1, ((0, 2), (0, 0)))
    h = _tc_mlp2(x8, w_e1p, b_e1, w_e2, b_e2, bn)

    rel_t = _sc_rel(pos_t, row3d, col3d0)
    frame_t = _tc_frame(rel_t)

    for lp in params["layers"]:
        (ws1, bs1), (ws2, bs2), (ws3, bs3) = lp["scalar"]
        (wv1, bv1), (wv2, bv2), (wv3, bv3) = lp["vector"]
        (wu1, bu1), (wu2, bu2) = lp["update"]

        wa = jnp.concatenate([ws1[:H], wv1[:H]], axis=1)
        wb = jnp.concatenate([ws1[H : 2 * H], wv1[H : 2 * H]], axis=1)
        wc4 = jnp.pad(
            jnp.concatenate([ws1[2 * H :], wv1[2 * H :]], axis=1), ((0, 1), (0, 0))
        )
        b1 = jnp.concatenate([bs1, bv1])
        w3a = jnp.pad(ws3, ((0, 0), (0, 3)))
        w3b = jnp.pad(wv3, ((0, 0), (1, 1)))
        b3 = jnp.concatenate([bs3, bv3, jnp.zeros((1,), _F32)])
        b3c = jnp.tile(b3.reshape(4, 1), (1, 128))
        wu1h = wu1[:H]
        wu1a = jnp.pad(wu1[H:], ((0, 1), (0, 0)))

        gr = _sc_gather_one(h, row3d)
        gc = _sc_gather_one(h, col3d0)
        coef_t = _tc_edge(
            gr, gc, frame_t, wa, wb, wc4, b1, ws2, bs2, wv2, bv2, w3a, w3b, b3c
        )
        aggr = _sc_scatter(coef_t, frame_t, col3dN, zeros4n)
        h = _tc_update(h, _tc_reduce_t(aggr), wu1h, wu1a, bu1, wu2, bu2, bn)

    (w_d1, b_d1), (w_d2, b_d2) = params["dec"]
    return _tc_mlp2(h, w_d1, b_d1, w_d2, b_d2, bn)


# speedup vs baseline: 1.0000x; 1.0000x over previous
import jax
import jax.numpy as jnp
import numpy as np
from jax import lax
from jax.experimental import pallas as pl
from jax.experimental.pallas import tpu as pltpu
from jax.experimental.pallas import tpu_sc as plsc

N = 10000
E = 320000
H = 128

NC = 2
NS = 16
NW = NC * NS
EW = E // NW
EWP = 10240
EP = NW * EWP
C = 128
NCH = EWP // C
GRP = 2
NGRP = NCH // GRP
GE = GRP * C
NP = 10112

_F32 = jnp.float32
_I32 = jnp.int32


def _mesh():
    return plsc.VectorSubcoreMesh(
        core_axis_name="c", subcore_axis_name="s", num_cores=NC, num_subcores=NS
    )


def _wid():
    return lax.axis_index("s") * NC + lax.axis_index("c")


_CP = pltpu.CompilerParams(needs_layout_passes=False)


def _sc_gather_one(table, idx3d):
    nbuf = 5

    def body(tab, idx, out, idxv, b0, b1, b2, b3, b4, *sems):
        bufs = (b0, b1, b2, b3, b4)
        gsems = sems[:nbuf]
        osems = sems[nbuf:]
        wid = _wid()
        pltpu.sync_copy(idx.at[wid], idxv)
        base = wid * EWP

        def drain(buf, sem):
            pltpu.make_async_copy(tab.at[pl.ds(0, C)], buf, sem).wait()

        for b in range(nbuf):
            pltpu.async_copy(tab.at[idxv.at[b]], bufs[b], gsems[b])

        def outer(r, carry):
            for b in range(nbuf):
                ci = r * nbuf + b
                drain(bufs[b], gsems[b])
                pltpu.async_copy(bufs[b], out.at[pl.ds(base + ci * C, C)], osems[b])
            for b in range(nbuf):
                nxt = (r + 1) * nbuf + b

                @pl.when(nxt < NCH)
                def _():
                    drain(bufs[b], osems[b])
                    pltpu.async_copy(tab.at[idxv.at[nxt]], bufs[b], gsems[b])

            return carry

        lax.fori_loop(0, NCH // nbuf, outer, 0)
        for b in range(nbuf):
            drain(bufs[b], osems[b])

    fn = pl.kernel(
        body,
        out_type=jax.ShapeDtypeStruct((EP, H), _F32),
        mesh=_mesh(),
        compiler_params=_CP,
        scratch_types=[pltpu.VMEM((NCH, C), _I32)]
        + [pltpu.VMEM((C, H), _F32)] * nbuf
        + [pltpu.SemaphoreType.DMA] * (2 * nbuf),
    )
    return fn(table, idx3d)


_RQ = 2560
_RQCH = _RQ // C


def _sc_rel(pos_t, row3d, col3d):
    def body(pt, ridx, cidx, out, tab, idxr, idxc, ob):
        wid = _wid()
        pltpu.sync_copy(pt, tab)
        pltpu.sync_copy(ridx.at[wid], idxr)
        pltpu.sync_copy(cidx.at[wid], idxc)

        zeros16 = jnp.zeros((16,), _F32)
        k3 = jnp.full((16,), 3, _I32)

        def quarter(q, carry):
            for l in range(_RQCH):
                i = q * _RQCH + l
                ii = jnp.full((16,), i, _I32)
                for j in range(C // 16):
                    r = lax.iota(_I32, 16) + j * 16
                    ir = plsc.load_gather(idxr, [ii, r])
                    ic = plsc.load_gather(idxc, [ii, r])
                    rl = r + l * C
                    for k in range(3):
                        kk = jnp.full((16,), k, _I32)
                        pk = plsc.load_gather(tab, [kk, ir])
                        qk = plsc.load_gather(tab, [kk, ic])
                        plsc.store_scatter(ob, [kk, rl], qk - pk)
                    plsc.store_scatter(ob, [k3, rl], zeros16)
            pltpu.sync_copy(ob, out.at[wid, :, pl.ds(q * _RQ, _RQ)])
            return carry

        lax.fori_loop(0, EWP // _RQ, quarter, 0)

    fn = pl.kernel(
        body,
        out_type=jax.ShapeDtypeStruct((NW, 4, EWP), _F32),
        mesh=_mesh(),
        compiler_params=_CP,
        scratch_types=[
            pltpu.VMEM((4, N), _F32),
            pltpu.VMEM((NCH, C), _I32),
            pltpu.VMEM((NCH, C), _I32),
            pltpu.VMEM((4, _RQ), _F32),
        ],
    )
    return fn(pos_t, row3d, col3d)


def _sc_scatter(coef_t, frame_t, col3d, zeros4n):
    def body(coef_h, frame_h, cidx, z4, out, idxb, cb, fb, acc):
        wid = _wid()
        pltpu.sync_copy(z4, acc)
        pltpu.sync_copy(cidx.at[wid], idxb)

        k0 = jnp.full((16,), 0, _I32)
        k1 = jnp.full((16,), 1, _I32)
        k2 = jnp.full((16,), 2, _I32)

        def chunk(i, carry):
            pltpu.sync_copy(coef_h.at[wid, :, pl.ds(i * C, C)], cb)
            pltpu.sync_copy(frame_h.at[wid, :, pl.ds(i * C, C)], fb)
            ii = jnp.full((16,), i, _I32)
            for j in range(C // 16):
                r = lax.iota(_I32, 16) + j * 16
                ids = plsc.load_gather(idxb, [ii, r])

                def ldf(k):
                    return plsc.load_gather(fb, [jnp.full((16,), k, _I32), r])

                fs = plsc.load_gather(cb, [k0, r])
                fv0 = plsc.load_gather(cb, [k1, r])
                fv1 = plsc.load_gather(cb, [k2, r])
                e1x, e1y, e1z = ldf(4), ldf(5), ldf(6)
                e2x, e2y = ldf(8), ldf(9)
                e3x, e3y, e3z = ldf(12), ldf(13), ldf(14)
                fx = fs * e1x + fv0 * e2x + fv1 * e3x
                fy = fs * e1y + fv0 * e2y + fv1 * e3y
                fz = fs * e1z + fv1 * e3z
                plsc.addupdate_scatter(acc, [k0, ids], fx)
                plsc.addupdate_scatter(acc, [k1, ids], fy)
                plsc.addupdate_scatter(acc, [k2, ids], fz)
            return carry

        lax.fori_loop(0, NCH, chunk, 0)
        pltpu.sync_copy(acc, out.at[wid])

    fn = pl.kernel(
        body,
        out_type=jax.ShapeDtypeStruct((NW, 4, NP), _F32),
        mesh=_mesh(),
        compiler_params=_CP,
        scratch_types=[
            pltpu.VMEM((NCH, C), _I32),
            pltpu.VMEM((4, C), _F32),
            pltpu.VMEM((16, C), _F32),
            pltpu.VMEM((4, NP), _F32),
        ],
    )
    return fn(coef_t, frame_t, col3d, zeros4n)


def _mlp2_body(x_ref, w1_ref, b1_ref, w2_ref, b2_ref, o_ref):
    t = jnp.dot(x_ref[...], w1_ref[...], preferred_element_type=_F32) + b1_ref[...]
    t = jnp.maximum(t, 0.0)
    o_ref[...] = jnp.dot(t, w2_ref[...], preferred_element_type=_F32) + b2_ref[...]


def _tc_mlp2(x, w1, b1, w2, b2, bn):
    n, fi = x.shape
    fh = w1.shape[1]
    fo = w2.shape[1]
    return pl.pallas_call(
        _mlp2_body,
        grid=(n // bn,),
        in_specs=[
            pl.BlockSpec((bn, fi), lambda i: (i, 0)),
            pl.BlockSpec((fi, fh), lambda i: (0, 0)),
            pl.BlockSpec((1, fh), lambda i: (0, 0)),
            pl.BlockSpec((fh, fo), lambda i: (0, 0)),
            pl.BlockSpec((1, fo), lambda i: (0, 0)),
        ],
        out_specs=pl.BlockSpec((bn, fo), lambda i: (i, 0)),
        out_shape=jax.ShapeDtypeStruct((n, fo), _F32),
    )(x, w1, b1.reshape(1, -1), w2, b2.reshape(1, -1))


_BC = 2560
_NB = EWP // _BC


def _frame_body(rel_ref, ones_ref, s2_ref, pt_ref, r1t_ref, r2t_ref, o_ref):
    f32 = _F32
    rel = rel_ref[0]
    ones44 = ones_ref[...]
    s2 = s2_ref[...]
    ptj = pt_ref[...]
    r1t = r1t_ref[...]
    r2t = r2t_ref[...]

    d2 = jnp.dot(ones44, rel * rel, preferred_element_type=f32)
    e1 = rel / (jnp.sqrt(d2) + 1e-6)
    n2 = jnp.dot(s2, e1 * e1, preferred_element_type=f32)
    e2 = jnp.dot(ptj, e1, preferred_element_type=f32) / (jnp.sqrt(n2) + 1e-6)
    e3 = jnp.dot(r1t, e1, preferred_element_type=f32) * jnp.dot(
        r2t, e2, preferred_element_type=f32
    ) - jnp.dot(r2t, e1, preferred_element_type=f32) * jnp.dot(
        r1t, e2, preferred_element_type=f32
    )
    o_ref[0] = jnp.concatenate([rel, e1, e2, e3], axis=0)


def _tc_frame(rel_t):
    ones44 = jnp.ones((4, 4), _F32)
    s2 = np.zeros((4, 4), np.float32)
    s2[:, 0] = 1.0
    s2[:, 1] = 1.0
    perm = np.zeros((4, 4), np.float32)
    perm[1, 0] = 1.0
    perm[0, 1] = -1.0
    r1 = np.zeros((4, 4), np.float32)
    r1[1, 0] = 1.0
    r1[2, 1] = 1.0
    r1[0, 2] = 1.0
    r2 = np.zeros((4, 4), np.float32)
    r2[2, 0] = 1.0
    r2[0, 1] = 1.0
    r2[1, 2] = 1.0
    full = lambda: pl.BlockSpec((4, 4), lambda i: (0, 0))
    return pl.pallas_call(
        _frame_body,
        grid=(NW * _NB,),
        in_specs=[
            pl.BlockSpec((1, 4, _BC), lambda i: (i // _NB, 0, i % _NB)),
            full(), full(), full(), full(), full(),
        ],
        out_specs=pl.BlockSpec((1, 16, _BC), lambda i: (i // _NB, 0, i % _NB)),
        out_shape=jax.ShapeDtypeStruct((NW, 16, EWP), _F32),
    )(
        rel_t, ones44, jnp.asarray(s2), jnp.asarray(perm.T),
        jnp.asarray(r1.T), jnp.asarray(r2.T),
    )


def _edge_body(
    gr_ref, gc_ref, fr_ref, wa_ref, wb_ref, wc_ref, b1_ref,
    w2s_ref, b2s_ref, w2v_ref, b2v_ref, w3a_ref, w3b_ref, b3_ref, o_ref
):
    rel_t = fr_ref[0, 0:4, :]
    t = (
        jnp.dot(gr_ref[...], wa_ref[...], preferred_element_type=_F32)
        + jnp.dot(gc_ref[...], wb_ref[...], preferred_element_type=_F32)
        + lax.dot_general(
            rel_t, wc_ref[...], (((0,), (0,)), ((), ())),
            preferred_element_type=_F32,
        )
        + b1_ref[...]
    )
    t = jnp.maximum(t, 0.0)
    s = jnp.maximum(
        jnp.dot(t[:, :H], w2s_ref[...], preferred_element_type=_F32) + b2s_ref[...], 0.0
    )
    v = jnp.maximum(
        jnp.dot(t[:, H:], w2v_ref[...], preferred_element_type=_F32) + b2v_ref[...], 0.0
    )
    ct = lax.dot_general(
        w3a_ref[...], s, (((0,), (1,)), ((), ())), preferred_element_type=_F32
    ) + lax.dot_general(
        w3b_ref[...], v, (((0,), (1,)), ((), ())), preferred_element_type=_F32
    )
    b3v = b3_ref[...]
    o_ref[0] = ct + b3v[:, 0:1]


def _tc_edge(gr, gc, frame_t, wa, wb, wc4, b1, w2s, b2s, w2v, b2v, w3a, w3b, b3c):
    full = lambda shape: pl.BlockSpec(shape, lambda i: (0,) * len(shape))
    return pl.pallas_call(
        _edge_body,
        grid=(NW * _NB,),
        in_specs=[
            pl.BlockSpec((_BC, H), lambda i: (i, 0)),
            pl.BlockSpec((_BC, H), lambda i: (i, 0)),
            pl.BlockSpec((1, 16, _BC), lambda i: (i // _NB, 0, i % _NB)),
            full((H, 2 * H)),
            full((H, 2 * H)),
            full((4, 2 * H)),
            full((1, 2 * H)),
            full((H, H)),
            full((1, H)),
            full((H, H)),
            full((1, H)),
            full((H, 4)),
            full((H, 4)),
            full((4, 128)),
        ],
        out_specs=pl.BlockSpec((1, 4, _BC), lambda i: (i // _NB, 0, i % _NB)),
        out_shape=jax.ShapeDtypeStruct((NW, 4, EWP), _F32),
    )(
        gr, gc, frame_t, wa, wb, wc4, b1.reshape(1, -1),
        w2s, b2s.reshape(1, -1), w2v, b2v.reshape(1, -1),
        w3a, w3b, b3c,
    )


def _reduce_body(ag_ref, i4_ref, o_ref):
    a = jnp.sum(ag_ref[...], axis=0)
    o_ref[...] = lax.dot_general(
        a, i4_ref[...], (((0,), (0,)), ((), ())), preferred_element_type=_F32
    )


def _tc_reduce_t(aggr):
    return pl.pallas_call(
        _reduce_body,
        grid=(1,),
        in_specs=[
            pl.BlockSpec((NW, 4, NP), lambda i: (0, 0, 0)),
            pl.BlockSpec((4, 4), lambda i: (0, 0)),
        ],
        out_specs=pl.BlockSpec((NP, 4), lambda i: (0, 0)),
        out_shape=jax.ShapeDtypeStruct((NP, 4), _F32),
    )(aggr, jnp.eye(4, dtype=_F32))


def _update_body(h_ref, ag_ref, wh_ref, wa_ref, b1_ref, w2_ref, b2_ref, o_ref):
    t = (
        jnp.dot(h_ref[...], wh_ref[...], preferred_element_type=_F32)
        + jnp.dot(ag_ref[...], wa_ref[...], preferred_element_type=_F32)
        + b1_ref[...]
    )
    t = jnp.maximum(t, 0.0)
    o_ref[...] = h_ref[...] + jnp.dot(t, w2_ref[...], preferred_element_type=_F32) + b2_ref[...]


def _tc_update(h, aggrt, wh, wa4, b1, w2, b2, bn):
    full = lambda shape: pl.BlockSpec(shape, lambda i: (0,) * len(shape))
    return pl.pallas_call(
        _update_body,
        grid=(N // bn,),
        in_specs=[
            pl.BlockSpec((bn, H), lambda i: (i, 0)),
            pl.BlockSpec((bn, 4), lambda i: (i, 0)),
            full((H, H)),
            full((4, H)),
            full((1, H)),
            full((H, H)),
            full((1, H)),
        ],
        out_specs=pl.BlockSpec((bn, H), lambda i: (i, 0)),
        out_shape=jax.ShapeDtypeStruct((N, H), _F32),
    )(h, aggrt, wh, wa4, b1.reshape(1, -1), w2, b2.reshape(1, -1))


_BN = 2000


def kernel(x, edge_index, pos, params):
    bn = _BN

    x8 = jnp.pad(x, ((0, 0), (0, 2)))
    pos_t = jnp.pad(pos.T, ((0, 1), (0, 0)))
    rowp = jnp.pad(edge_index[0].reshape(NW, EW), ((0, 0), (0, EWP - EW)))
    colw = edge_index[1].reshape(NW, EW)
    row3d = rowp.reshape(NW, NCH, C)
    col3d0 = jnp.pad(colw, ((0, 0), (0, EWP - EW))).reshape(NW, NCH, C)
    col3dN = jnp.pad(
        colw, ((0, 0), (0, EWP - EW)), constant_values=N
    ).reshape(NW, NCH, C)
    zeros4n = jnp.zeros((4, NP), _F32)

    (w_e1, b_e1), (w_e2, b_e2) = params["enc"]
    w_e1p = jnp.pad(w_e1, ((0, 2), (0, 0)))
    h = _tc_mlp2(x8, w_e1p, b_e1, w_e2, b_e2, bn)

    rel_t = _sc_rel(pos_t, row3d, col3d0)
    frame_t = _tc_frame(rel_t)

    for lp in params["layers"]:
        (ws1, bs1), (ws2, bs2), (ws3, bs3) = lp["scalar"]
        (wv1, bv1), (wv2, bv2), (wv3, bv3) = lp["vector"]
        (wu1, bu1), (wu2, bu2) = lp["update"]

        wa = jnp.concatenate([ws1[:H], wv1[:H]], axis=1)
        wb = jnp.concatenate([ws1[H : 2 * H], wv1[H : 2 * H]], axis=1)
        wc4 = jnp.pad(
            jnp.concatenate([ws1[2 * H :], wv1[2 * H :]], axis=1), ((0, 1), (0, 0))
        )
        b1 = jnp.concatenate([bs1, bv1])
        w3a = jnp.pad(ws3, ((0, 0), (0, 3)))
        w3b = jnp.pad(wv3, ((0, 0), (1, 1)))
        b3 = jnp.concatenate([bs3, bv3, jnp.zeros((1,), _F32)])
        b3c = jnp.tile(b3.reshape(4, 1), (1, 128))
        wu1h = wu1[:H]
        wu1a = jnp.pad(wu1[H:], ((0, 1), (0, 0)))

        gr = _sc_gather_one(h, row3d)
        gc = _sc_gather_one(h, col3d0)
        coef_t = _tc_edge(
            gr, gc, frame_t, wa, wb, wc4, b1, ws2, bs2, wv2, bv2, w3a, w3b, b3c
        )
        aggr = _sc_scatter(coef_t, frame_t, col3dN, zeros4n)
        h = _tc_update(h, _tc_reduce_t(aggr), wu1h, wu1a, bu1, wu2, bu2, bn)

    (w_d1, b_d1), (w_d2, b_d2) = params["dec"]
    return _tc_mlp2(h, w_d1, b_d1, w_d2, b_d2, bn)


# speedup vs baseline: 1.1059x; 1.1059x over previous
import jax
import jax.numpy as jnp
import numpy as np
from jax import lax
from jax.experimental import pallas as pl
from jax.experimental.pallas import tpu as pltpu
from jax.experimental.pallas import tpu_sc as plsc

N = 10000
E = 320000
H = 128

NC = 2
NS = 16
NW = NC * NS
EW = E // NW
EWP = 10240
EP = NW * EWP
C = 128
NCH = EWP // C
GRP = 2
NGRP = NCH // GRP
GE = GRP * C
NP = 10112

_F32 = jnp.float32
_I32 = jnp.int32


def _mesh():
    return plsc.VectorSubcoreMesh(
        core_axis_name="c", subcore_axis_name="s", num_cores=NC, num_subcores=NS
    )


def _wid():
    return lax.axis_index("s") * NC + lax.axis_index("c")


_CP = pltpu.CompilerParams(needs_layout_passes=False)


def _sc_gather(table, idx3d):
    nbuf = 5
    dt = table.dtype
    nch = idx3d.shape[1]
    ewp = nch * C

    def body(tab, idx, out, idxv, b0, b1, b2, b3, b4, *sems):
        bufs = (b0, b1, b2, b3, b4)
        gsems = sems[:nbuf]
        osems = sems[nbuf:]
        wid = _wid()
        pltpu.sync_copy(idx.at[wid], idxv)
        base = wid * ewp

        def drain(buf, sem):
            pltpu.make_async_copy(tab.at[pl.ds(0, C)], buf, sem).wait()

        for b in range(nbuf):
            pltpu.async_copy(tab.at[idxv.at[b]], bufs[b], gsems[b])

        def outer(r, carry):
            for b in range(nbuf):
                ci = r * nbuf + b
                drain(bufs[b], gsems[b])
                pltpu.async_copy(bufs[b], out.at[pl.ds(base + ci * C, C)], osems[b])
            for b in range(nbuf):
                nxt = (r + 1) * nbuf + b

                @pl.when(nxt < nch)
                def _():
                    drain(bufs[b], osems[b])
                    pltpu.async_copy(tab.at[idxv.at[nxt]], bufs[b], gsems[b])

            return carry

        lax.fori_loop(0, nch // nbuf, outer, 0)
        for b in range(nbuf):
            drain(bufs[b], osems[b])

    fn = pl.kernel(
        body,
        out_type=jax.ShapeDtypeStruct((NW * ewp, H), dt),
        mesh=_mesh(),
        compiler_params=_CP,
        scratch_types=[pltpu.VMEM((nch, C), _I32)]
        + [pltpu.VMEM((C, H), dt)] * nbuf
        + [pltpu.SemaphoreType.DMA] * (2 * nbuf),
    )
    return fn(table, idx3d)


_RQ = 2560
_RQCH = _RQ // C


def _sc_rel(pos_t, row3d, col3d):
    def body(pt, ridx, cidx, out, tab, idxr, idxc, ob):
        wid = _wid()
        pltpu.sync_copy(pt, tab)
        pltpu.sync_copy(ridx.at[wid], idxr)
        pltpu.sync_copy(cidx.at[wid], idxc)

        zeros16 = jnp.zeros((16,), _F32)
        k3 = jnp.full((16,), 3, _I32)

        def quarter(q, carry):
            for l in range(_RQCH):
                i = q * _RQCH + l
                ii = jnp.full((16,), i, _I32)
                for j in range(C // 16):
                    r = lax.iota(_I32, 16) + j * 16
                    ir = plsc.load_gather(idxr, [ii, r])
                    ic = plsc.load_gather(idxc, [ii, r])
                    rl = r + l * C
                    for k in range(3):
                        kk = jnp.full((16,), k, _I32)
                        pk = plsc.load_gather(tab, [kk, ir])
                        qk = plsc.load_gather(tab, [kk, ic])
                        plsc.store_scatter(ob, [kk, rl], qk - pk)
                    plsc.store_scatter(ob, [k3, rl], zeros16)
            pltpu.sync_copy(ob, out.at[wid, :, pl.ds(q * _RQ, _RQ)])
            return carry

        lax.fori_loop(0, EWP // _RQ, quarter, 0)

    fn = pl.kernel(
        body,
        out_type=jax.ShapeDtypeStruct((NW, 4, EWP), _F32),
        mesh=_mesh(),
        compiler_params=_CP,
        scratch_types=[
            pltpu.VMEM((4, N), _F32),
            pltpu.VMEM((NCH, C), _I32),
            pltpu.VMEM((NCH, C), _I32),
            pltpu.VMEM((4, _RQ), _F32),
        ],
    )
    return fn(pos_t, row3d, col3d)


def _sc_scatter(coef_t, frame_t, col3d, zeros4n, off):
    nch = col3d.shape[1]

    def body(coef_h, frame_h, cidx, z4, out, idxb, cb, fb, acc):
        wid = _wid()
        pltpu.sync_copy(z4, acc)
        pltpu.sync_copy(cidx.at[wid], idxb)

        k0 = jnp.full((16,), 0, _I32)
        k1 = jnp.full((16,), 1, _I32)
        k2 = jnp.full((16,), 2, _I32)

        def chunk(i, carry):
            pltpu.sync_copy(coef_h.at[wid, :, pl.ds(i * C, C)], cb)
            pltpu.sync_copy(frame_h.at[wid, :, pl.ds((off + i) * C, C)], fb)
            ii = jnp.full((16,), i, _I32)
            for j in range(C // 16):
                r = lax.iota(_I32, 16) + j * 16
                ids = plsc.load_gather(idxb, [ii, r])

                def ldf(k):
                    return plsc.load_gather(fb, [jnp.full((16,), k, _I32), r])

                fs = plsc.load_gather(cb, [k0, r])
                fv0 = plsc.load_gather(cb, [k1, r])
                fv1 = plsc.load_gather(cb, [k2, r])
                e1x, e1y, e1z = ldf(4), ldf(5), ldf(6)
                e2x, e2y = ldf(8), ldf(9)
                e3x, e3y, e3z = ldf(12), ldf(13), ldf(14)
                fx = fs * e1x + fv0 * e2x + fv1 * e3x
                fy = fs * e1y + fv0 * e2y + fv1 * e3y
                fz = fs * e1z + fv1 * e3z
                plsc.addupdate_scatter(acc, [k0, ids], fx)
                plsc.addupdate_scatter(acc, [k1, ids], fy)
                plsc.addupdate_scatter(acc, [k2, ids], fz)
            return carry

        lax.fori_loop(0, nch, chunk, 0)
        pltpu.sync_copy(acc, out.at[wid])

    fn = pl.kernel(
        body,
        out_type=jax.ShapeDtypeStruct((NW, 4, NP), _F32),
        mesh=_mesh(),
        compiler_params=_CP,
        scratch_types=[
            pltpu.VMEM((nch, C), _I32),
            pltpu.VMEM((4, C), _F32),
            pltpu.VMEM((16, C), _F32),
            pltpu.VMEM((4, NP), _F32),
        ],
    )
    return fn(coef_t, frame_t, col3d, zeros4n)


def _mlp2_body(x_ref, w1_ref, b1_ref, w2_ref, b2_ref, o_ref):
    t = jnp.dot(x_ref[...], w1_ref[...], preferred_element_type=_F32) + b1_ref[...]
    t = jnp.maximum(t, 0.0)
    o_ref[...] = jnp.dot(t, w2_ref[...], preferred_element_type=_F32) + b2_ref[...]


def _tc_mlp2(x, w1, b1, w2, b2, bn):
    n, fi = x.shape
    fh = w1.shape[1]
    fo = w2.shape[1]
    return pl.pallas_call(
        _mlp2_body,
        grid=(n // bn,),
        in_specs=[
            pl.BlockSpec((bn, fi), lambda i: (i, 0)),
            pl.BlockSpec((fi, fh), lambda i: (0, 0)),
            pl.BlockSpec((1, fh), lambda i: (0, 0)),
            pl.BlockSpec((fh, fo), lambda i: (0, 0)),
            pl.BlockSpec((1, fo), lambda i: (0, 0)),
        ],
        out_specs=pl.BlockSpec((bn, fo), lambda i: (i, 0)),
        out_shape=jax.ShapeDtypeStruct((n, fo), _F32),
    )(x, w1, b1.reshape(1, -1), w2, b2.reshape(1, -1))


_BC = 2560
_NB = EWP // _BC


def _frame_body(rel_ref, ones_ref, s2_ref, pt_ref, r1t_ref, r2t_ref, o_ref):
    f32 = _F32
    rel = rel_ref[0]
    ones44 = ones_ref[...]
    s2 = s2_ref[...]
    ptj = pt_ref[...]
    r1t = r1t_ref[...]
    r2t = r2t_ref[...]

    d2 = jnp.dot(ones44, rel * rel, preferred_element_type=f32)
    e1 = rel / (jnp.sqrt(d2) + 1e-6)
    n2 = jnp.dot(s2, e1 * e1, preferred_element_type=f32)
    e2 = jnp.dot(ptj, e1, preferred_element_type=f32) / (jnp.sqrt(n2) + 1e-6)
    e3 = jnp.dot(r1t, e1, preferred_element_type=f32) * jnp.dot(
        r2t, e2, preferred_element_type=f32
    ) - jnp.dot(r2t, e1, preferred_element_type=f32) * jnp.dot(
        r1t, e2, preferred_element_type=f32
    )
    o_ref[0] = jnp.concatenate([rel, e1, e2, e3], axis=0)


def _tc_frame(rel_t):
    ones44 = jnp.ones((4, 4), _F32)
    s2 = np.zeros((4, 4), np.float32)
    s2[:, 0] = 1.0
    s2[:, 1] = 1.0
    perm = np.zeros((4, 4), np.float32)
    perm[1, 0] = 1.0
    perm[0, 1] = -1.0
    r1 = np.zeros((4, 4), np.float32)
    r1[1, 0] = 1.0
    r1[2, 1] = 1.0
    r1[0, 2] = 1.0
    r2 = np.zeros((4, 4), np.float32)
    r2[2, 0] = 1.0
    r2[0, 1] = 1.0
    r2[1, 2] = 1.0
    full = lambda: pl.BlockSpec((4, 4), lambda i: (0, 0))
    return pl.pallas_call(
        _frame_body,
        grid=(NW * _NB,),
        in_specs=[
            pl.BlockSpec((1, 4, _BC), lambda i: (i // _NB, 0, i % _NB)),
            full(), full(), full(), full(), full(),
        ],
        out_specs=pl.BlockSpec((1, 16, _BC), lambda i: (i // _NB, 0, i % _NB)),
        out_shape=jax.ShapeDtypeStruct((NW, 16, EWP), _F32),
    )(
        rel_t, ones44, jnp.asarray(s2), jnp.asarray(perm.T),
        jnp.asarray(r1.T), jnp.asarray(r2.T),
    )


def _edge_body(
    gr_ref, gc_ref, fr_ref, wa_ref, wb_ref, wc_ref, b1_ref,
    w2s_ref, b2s_ref, w2v_ref, b2v_ref, w3a_ref, w3b_ref, b3_ref, o_ref
):
    rel_t = fr_ref[0, 0:4, :]
    t = (
        jnp.dot(gr_ref[...].astype(_F32), wa_ref[...], preferred_element_type=_F32)
        + jnp.dot(gc_ref[...].astype(_F32), wb_ref[...], preferred_element_type=_F32)
        + lax.dot_general(
            rel_t, wc_ref[...], (((0,), (0,)), ((), ())),
            preferred_element_type=_F32,
        )
        + b1_ref[...]
    )
    t = jnp.maximum(t, 0.0)
    s = jnp.maximum(
        jnp.dot(t[:, :H], w2s_ref[...], preferred_element_type=_F32) + b2s_ref[...], 0.0
    )
    v = jnp.maximum(
        jnp.dot(t[:, H:], w2v_ref[...], preferred_element_type=_F32) + b2v_ref[...], 0.0
    )
    ct = lax.dot_general(
        w3a_ref[...], s, (((0,), (1,)), ((), ())), preferred_element_type=_F32
    ) + lax.dot_general(
        w3b_ref[...], v, (((0,), (1,)), ((), ())), preferred_element_type=_F32
    )
    b3v = b3_ref[...]
    o_ref[0] = ct + b3v[:, 0:1]


def _tc_edge(
    gr, gc, frame_t, wa, wb, wc4, b1, w2s, b2s, w2v, b2v, w3a, w3b, b3c, boff, nb
):
    full = lambda shape: pl.BlockSpec(shape, lambda i: (0,) * len(shape))
    return pl.pallas_call(
        _edge_body,
        grid=(NW * nb,),
        in_specs=[
            pl.BlockSpec((_BC, H), lambda i: (i, 0)),
            pl.BlockSpec((_BC, H), lambda i: (i, 0)),
            pl.BlockSpec((1, 16, _BC), lambda i: (i // nb, 0, boff + i % nb)),
            full((H, 2 * H)),
            full((H, 2 * H)),
            full((4, 2 * H)),
            full((1, 2 * H)),
            full((H, H)),
            full((1, H)),
            full((H, H)),
            full((1, H)),
            full((H, 4)),
            full((H, 4)),
            full((4, 128)),
        ],
        out_specs=pl.BlockSpec((1, 4, _BC), lambda i: (i // nb, 0, i % nb)),
        out_shape=jax.ShapeDtypeStruct((NW, 4, nb * _BC), _F32),
    )(
        gr, gc, frame_t, wa, wb, wc4, b1.reshape(1, -1),
        w2s, b2s.reshape(1, -1), w2v, b2v.reshape(1, -1),
        w3a, w3b, b3c,
    )


def _reduce_body(aga_ref, agb_ref, i4_ref, o_ref):
    a = jnp.sum(aga_ref[...], axis=0) + jnp.sum(agb_ref[...], axis=0)
    o_ref[...] = lax.dot_general(
        a, i4_ref[...], (((0,), (0,)), ((), ())), preferred_element_type=_F32
    )


def _tc_reduce_t(aggr_a, aggr_b):
    return pl.pallas_call(
        _reduce_body,
        grid=(1,),
        in_specs=[
            pl.BlockSpec((NW, 4, NP), lambda i: (0, 0, 0)),
            pl.BlockSpec((NW, 4, NP), lambda i: (0, 0, 0)),
            pl.BlockSpec((4, 4), lambda i: (0, 0)),
        ],
        out_specs=pl.BlockSpec((NP, 4), lambda i: (0, 0)),
        out_shape=jax.ShapeDtypeStruct((NP, 4), _F32),
    )(aggr_a, aggr_b, jnp.eye(4, dtype=_F32))


def _update_body(h_ref, ag_ref, wh_ref, wa_ref, b1_ref, w2_ref, b2_ref, o_ref):
    t = (
        jnp.dot(h_ref[...], wh_ref[...], preferred_element_type=_F32)
        + jnp.dot(ag_ref[...], wa_ref[...], preferred_element_type=_F32)
        + b1_ref[...]
    )
    t = jnp.maximum(t, 0.0)
    o_ref[...] = h_ref[...] + jnp.dot(t, w2_ref[...], preferred_element_type=_F32) + b2_ref[...]


def _tc_update(h, aggrt, wh, wa4, b1, w2, b2, bn):
    full = lambda shape: pl.BlockSpec(shape, lambda i: (0,) * len(shape))
    return pl.pallas_call(
        _update_body,
        grid=(N // bn,),
        in_specs=[
            pl.BlockSpec((bn, H), lambda i: (i, 0)),
            pl.BlockSpec((bn, 4), lambda i: (i, 0)),
            full((H, H)),
            full((4, H)),
            full((1, H)),
            full((H, H)),
            full((1, H)),
        ],
        out_specs=pl.BlockSpec((bn, H), lambda i: (i, 0)),
        out_shape=jax.ShapeDtypeStruct((N, H), _F32),
    )(h, aggrt, wh, wa4, b1.reshape(1, -1), w2, b2.reshape(1, -1))


_BN = 2000


def kernel(x, edge_index, pos, params):
    bn = _BN

    x8 = jnp.pad(x, ((0, 0), (0, 2)))
    pos_t = jnp.pad(pos.T, ((0, 1), (0, 0)))
    rowp = jnp.pad(edge_index[0].reshape(NW, EW), ((0, 0), (0, EWP - EW)))
    colw = edge_index[1].reshape(NW, EW)
    row3d = rowp.reshape(NW, NCH, C)
    col3d0 = jnp.pad(colw, ((0, 0), (0, EWP - EW))).reshape(NW, NCH, C)
    col3dN = jnp.pad(
        colw, ((0, 0), (0, EWP - EW)), constant_values=N
    ).reshape(NW, NCH, C)
    zeros4n = jnp.zeros((4, NP), _F32)
    nch2 = NCH // 2
    nb2 = _NB // 2
    rowA, rowB = row3d[:, :nch2], row3d[:, nch2:]
    colA0, colB0 = col3d0[:, :nch2], col3d0[:, nch2:]
    colAN, colBN = col3dN[:, :nch2], col3dN[:, nch2:]

    (w_e1, b_e1), (w_e2, b_e2) = params["enc"]
    w_e1p = jnp.pad(w_e1, ((0, 2), (0, 0)))
    h = _tc_mlp2(x8, w_e1p, b_e1, w_e2, b_e2, bn)

    rel_t = _sc_rel(pos_t, row3d, col3d0)
    frame_t = _tc_frame(rel_t)

    for lp in params["layers"]:
        (ws1, bs1), (ws2, bs2), (ws3, bs3) = lp["scalar"]
        (wv1, bv1), (wv2, bv2), (wv3, bv3) = lp["vector"]
        (wu1, bu1), (wu2, bu2) = lp["update"]

        wa = jnp.concatenate([ws1[:H], wv1[:H]], axis=1)
        wb = jnp.concatenate([ws1[H : 2 * H], wv1[H : 2 * H]], axis=1)
        wc4 = jnp.pad(
            jnp.concatenate([ws1[2 * H :], wv1[2 * H :]], axis=1), ((0, 1), (0, 0))
        )
        b1 = jnp.concatenate([bs1, bv1])
        w3a = jnp.pad(ws3, ((0, 0), (0, 3)))
        w3b = jnp.pad(wv3, ((0, 0), (1, 1)))
        b3 = jnp.concatenate([bs3, bv3, jnp.zeros((1,), _F32)])
        b3c = jnp.tile(b3.reshape(4, 1), (1, 128))
        wu1h = wu1[:H]
        wu1a = jnp.pad(wu1[H:], ((0, 1), (0, 0)))

        grA = _sc_gather(h, rowA)
        gcA = _sc_gather(h, colA0)
        grB = _sc_gather(h, rowB)
        gcB = _sc_gather(h, colB0)
        coefA = _tc_edge(
            grA, gcA, frame_t, wa, wb, wc4, b1, ws2, bs2, wv2, bv2,
            w3a, w3b, b3c, 0, nb2,
        )
        coefB = _tc_edge(
            grB, gcB, frame_t, wa, wb, wc4, b1, ws2, bs2, wv2, bv2,
            w3a, w3b, b3c, nb2, nb2,
        )
        aggrA = _sc_scatter(coefA, frame_t, colAN, zeros4n, 0)
        aggrB = _sc_scatter(coefB, frame_t, colBN, zeros4n, nch2)
        h = _tc_update(h, _tc_reduce_t(aggrA, aggrB), wu1h, wu1a, bu1, wu2, bu2, bn)

    (w_d1, b_d1), (w_d2, b_d2) = params["dec"]
    return _tc_mlp2(h, w_d1, b_d1, w_d2, b_d2, bn)
